# Initial kernel scaffold; baseline (speedup 1.0000x reference)
import jax
import jax.numpy as jnp
from jax.experimental import pallas as pl


def kernel(x, gate_weight, Wg, Wu, Wd):
    raise NotImplementedError("write your pallas kernel here")


# speedup vs baseline: 1.7726x; 1.7726x over previous
import functools

import jax
import jax.numpy as jnp
from jax import lax
from jax.experimental import pallas as pl
from jax.experimental.pallas import tpu as pltpu

B, S, D, E, FF = 2, 2048, 1024, 8, 2048
K = 512
SB = 512
FFB = 512


def _gating_body(x_ref, gw_ref, out_ref):
    logits_t = lax.dot_general(
        gw_ref[...], x_ref[0], (((0,), (1,)), ((), ())),
        preferred_element_type=jnp.float32)
    m = jnp.max(logits_t, axis=0, keepdims=True)
    ex = jnp.exp(logits_t - m)
    out_ref[0] = ex / jnp.sum(ex, axis=0, keepdims=True)


def _gating(x, gate_weight):
    return pl.pallas_call(
        _gating_body,
        grid=(B, S // SB),
        in_specs=[
            pl.BlockSpec((1, SB, D), lambda b, s: (b, s, 0)),
            pl.BlockSpec((D, E), lambda b, s: (0, 0)),
        ],
        out_specs=pl.BlockSpec((1, E, SB), lambda b, s: (b, 0, s)),
        out_shape=jax.ShapeDtypeStruct((B, E, S), jnp.float32),
    )(x, gate_weight)


def _swiglu_body(xsel_ref, gates_ref, wg_ref, wu_ref, wd_ref, out_ref):
    a = xsel_ref[0]
    g = lax.dot_general(a, wg_ref[0], (((1,), (1,)), ((), ())),
                        preferred_element_type=jnp.float32)
    u = lax.dot_general(a, wu_ref[0], (((1,), (1,)), ((), ())),
                        preferred_element_type=jnp.float32)
    h = g * jax.nn.sigmoid(g) * u
    h = h * gates_ref[0, 0][:, None]
    part = lax.dot_general(h, wd_ref[0], (((1,), (1,)), ((), ())),
                           preferred_element_type=jnp.float32)

    @pl.when(pl.program_id(1) == 0)
    def _():
        out_ref[0] = part

    @pl.when(pl.program_id(1) != 0)
    def _():
        out_ref[0] += part


def _swiglu(x_sel, gates, Wg, Wu, Wd):
    return pl.pallas_call(
        _swiglu_body,
        grid=(E, FF // FFB),
        in_specs=[
            pl.BlockSpec((1, B * K, D), lambda e, f: (e, 0, 0)),
            pl.BlockSpec((1, 1, B * K), lambda e, f: (e, 0, 0)),
            pl.BlockSpec((1, FFB, D), lambda e, f: (e, f, 0)),
            pl.BlockSpec((1, FFB, D), lambda e, f: (e, f, 0)),
            pl.BlockSpec((1, D, FFB), lambda e, f: (e, 0, f)),
        ],
        out_specs=pl.BlockSpec((1, B * K, D), lambda e, f: (e, 0, 0)),
        out_shape=jax.ShapeDtypeStruct((E, B * K, D), jnp.float32),
    )(x_sel, gates, Wg, Wu, Wd)


def kernel(x, gate_weight, Wg, Wu, Wd):
    aff_t = _gating(x, gate_weight)
    gating, index = lax.top_k(aff_t, K)
    x2d = x.reshape(B * S, D)
    gidx = (index + jnp.arange(B)[:, None, None] * S)
    gidx_eb = jnp.transpose(gidx, (1, 0, 2)).reshape(E, B * K)
    gates_eb = jnp.transpose(gating, (1, 0, 2)).reshape(E, 1, B * K)
    x_sel = x2d[gidx_eb.reshape(-1)].reshape(E, B * K, D)
    contrib = _swiglu(x_sel, gates_eb, Wg, Wu, Wd)
    out = jnp.zeros((B * S, D), jnp.float32).at[gidx_eb.reshape(-1)].add(
        contrib.reshape(-1, D))
    return out.reshape(B, S, D)
